# Initial kernel scaffold; baseline (speedup 1.0000x reference)
import jax
import jax.numpy as jnp
from jax.experimental import pallas as pl


def kernel(x, Wg, bg, W_experts, b_experts, Wp, bp, noise):
    raise NotImplementedError("write your pallas kernel here")


# speedup vs baseline: 4.5137x; 4.5137x over previous
import jax
import jax.numpy as jnp
from jax import lax
from jax.experimental import pallas as pl
from jax.experimental.pallas import tpu as pltpu

N = 4096
D = 768
E = 8
F = 768
TB = 512


def _moe_block(x_ref, wg_ref, bg_ref, noise_ref, we_ref, be_ref, wp_ref, bp_ref,
               out_ref):
    x = x_ref[...]
    s = lax.dot_general(x, wg_ref[...], (((1,), (1,)), ((), ())),
                        preferred_element_type=jnp.float32)
    s = s + bg_ref[...] + noise_ref[...]
    m = jnp.max(s, axis=1, keepdims=True)
    p = jnp.exp(s - m)
    gw = p / jnp.sum(p, axis=1, keepdims=True)
    wt = jnp.max(gw, axis=1, keepdims=True)
    ii = lax.broadcasted_iota(jnp.int32, (TB, E), 1)
    eid = jnp.min(jnp.where(gw == wt, ii, E), axis=1, keepdims=True)

    hsel = jnp.zeros((TB, F), jnp.float32)
    for ex in range(E):
        h = lax.dot_general(x, we_ref[ex], (((1,), (1,)), ((), ())),
                            preferred_element_type=jnp.float32)
        h = h + be_ref[ex][None, :]
        hsel = hsel + jnp.where(eid == ex, h, 0.0)

    y = lax.dot_general(hsel, wp_ref[...], (((1,), (1,)), ((), ())),
                        preferred_element_type=jnp.float32)
    y = y + bp_ref[...]
    out_ref[...] = wt * y


def kernel(x, Wg, bg, W_experts, b_experts, Wp, bp, noise):
    orig_shape = x.shape
    x_flat = x.reshape(N, D)
    grid = (N // TB,)
    out = pl.pallas_call(
        _moe_block,
        grid=grid,
        in_specs=[
            pl.BlockSpec((TB, D), lambda i: (i, 0)),
            pl.BlockSpec((E, D), lambda i: (0, 0)),
            pl.BlockSpec((1, E), lambda i: (0, 0)),
            pl.BlockSpec((TB, E), lambda i: (i, 0)),
            pl.BlockSpec((E, F, D), lambda i: (0, 0, 0)),
            pl.BlockSpec((E, F), lambda i: (0, 0)),
            pl.BlockSpec((D, F), lambda i: (0, 0)),
            pl.BlockSpec((1, D), lambda i: (0, 0)),
        ],
        out_specs=pl.BlockSpec((TB, D), lambda i: (i, 0)),
        out_shape=jax.ShapeDtypeStruct((N, D), jnp.float32),
    )(x_flat, Wg, bg.reshape(1, E), noise, W_experts, b_experts, Wp,
      bp.reshape(1, D))
    return out.reshape(orig_shape)
